# Initial kernel scaffold; baseline (speedup 1.0000x reference)
import jax
import jax.numpy as jnp
from jax.experimental import pallas as pl


def kernel(x, edge_index, W1, b1, W2, b2):
    raise NotImplementedError("write your pallas kernel here")


# speedup vs baseline: 35.4284x; 35.4284x over previous
import functools

import jax
import jax.numpy as jnp
from jax import lax
from jax.experimental import pallas as pl
from jax.experimental.pallas import tpu as pltpu
from jax.experimental.pallas import tpu_sc as plsc

N_NODES = 10000
N_PAD = 10240
N_EDGES = 320000
D_IN = 128
D_HID = 16

NC = 2
NS = 16
NW = NC * NS
E_W = N_EDGES // NW
K = 125
WIN = E_W // K
ROWS_W = N_PAD // NS

_MESH = plsc.VectorSubcoreMesh(core_axis_name="c", subcore_axis_name="s")
_SC_PARAMS = pltpu.CompilerParams(use_tc_tiling_on_sc=False)


@functools.partial(
    pl.kernel,
    out_type=jax.ShapeDtypeStruct((NC, N_PAD), jnp.float32),
    mesh=_MESH,
    scratch_types=[
        pltpu.VMEM((WIN, K), jnp.int32),
        pltpu.VMEM((K,), jnp.float32),
        pltpu.VMEM_SHARED((N_PAD,), jnp.float32),
    ],
    compiler_params=_SC_PARAMS,
)
def _sc_degree(dst_hbm, ones_hbm, zeros_hbm, out_hbm, idx_v, ones_v, acc_sh):
    c = lax.axis_index("c")
    s = lax.axis_index("s")
    w = c * NS + s
    pltpu.sync_copy(ones_hbm, ones_v)
    pltpu.sync_copy(zeros_hbm.at[pl.ds(0, ROWS_W)],
                    acc_sh.at[pl.ds(s * ROWS_W, ROWS_W)])
    plsc.subcore_barrier()
    pltpu.sync_copy(dst_hbm.at[pl.ds(w * WIN, WIN)], idx_v)

    @pl.loop(0, WIN)
    def _(j):
        pltpu.sync_copy(ones_v, acc_sh.at[idx_v.at[j]], add=True)

    plsc.subcore_barrier()
    pltpu.sync_copy(acc_sh.at[pl.ds(s * ROWS_W, ROWS_W)],
                    out_hbm.at[c].at[pl.ds(s * ROWS_W, ROWS_W)])


@functools.partial(
    pl.kernel,
    out_type=jax.ShapeDtypeStruct((NC, N_PAD, D_HID), jnp.float32),
    mesh=_MESH,
    scratch_types=[
        pltpu.VMEM((WIN, K), jnp.int32),
        pltpu.VMEM((WIN, K), jnp.int32),
        pltpu.VMEM((K, D_HID), jnp.float32),
        pltpu.VMEM_SHARED((N_PAD, D_HID), jnp.float32),
    ],
    compiler_params=_SC_PARAMS,
)
def _sc_aggregate(table_hbm, src_hbm, dst_hbm, zeros_hbm, out_hbm,
                  src_v, dst_v, rows_v, acc_sh):
    c = lax.axis_index("c")
    s = lax.axis_index("s")
    w = c * NS + s
    pltpu.sync_copy(zeros_hbm, acc_sh.at[pl.ds(s * ROWS_W, ROWS_W)])
    plsc.subcore_barrier()
    pltpu.sync_copy(src_hbm.at[pl.ds(w * WIN, WIN)], src_v)
    pltpu.sync_copy(dst_hbm.at[pl.ds(w * WIN, WIN)], dst_v)

    @pl.loop(0, WIN)
    def _(j):
        pltpu.sync_copy(table_hbm.at[src_v.at[j]], rows_v)
        pltpu.sync_copy(rows_v, acc_sh.at[dst_v.at[j]], add=True)

    plsc.subcore_barrier()
    pltpu.sync_copy(acc_sh.at[pl.ds(s * ROWS_W, ROWS_W)],
                    out_hbm.at[c].at[pl.ds(s * ROWS_W, ROWS_W)])


def _tc_xw_body(x_ref, w1_ref, xw_ref):
    xw_ref[...] = jnp.dot(x_ref[...], w1_ref[...],
                          preferred_element_type=jnp.float32)


def _tc_prescale_body(deg_ref, xw_ref, dinv_ref, xws_ref):
    deg = deg_ref[0:1, :] + deg_ref[1:2, :] + 1.0
    dinv = 1.0 / jnp.sqrt(deg)
    dcol = dinv.reshape(N_PAD, 1)
    dinv_ref[...] = dcol
    xws_ref[...] = xw_ref[...] * dcol[0:N_NODES, :]


def _tc_mid_body(p_ref, xw_ref, dinv_ref, b1_ref, h_ref, hs_ref):
    dc = dinv_ref[0:N_NODES, :]
    agg = p_ref[0, 0:N_NODES, :] + p_ref[1, 0:N_NODES, :]
    pre = dc * agg + dc * dc * xw_ref[...] + b1_ref[...]
    h = jnp.maximum(pre, 0.0)
    h_ref[...] = h
    hs_ref[...] = h * dc


def _tc_final_body(q_ref, h_ref, dinv_ref, w2_ref, b2_ref, out_ref):
    dc = dinv_ref[0:N_NODES, :]
    agg2 = q_ref[0, 0:N_NODES, :] + q_ref[1, 0:N_NODES, :]
    z = dc * agg2 + dc * dc * h_ref[...]
    o = jnp.dot(z, w2_ref[...], preferred_element_type=jnp.float32) + b2_ref[...]
    m = jnp.max(o, axis=1, keepdims=True)
    lse = m + jnp.log(jnp.sum(jnp.exp(o - m), axis=1, keepdims=True))
    out_ref[...] = o - lse


def _f32(shape):
    return jax.ShapeDtypeStruct(shape, jnp.float32)


def kernel(x, edge_index, W1, b1, W2, b2):
    src = edge_index[0].astype(jnp.int32).reshape(NW * WIN, K)
    dst = edge_index[1].astype(jnp.int32).reshape(NW * WIN, K)
    ones_k = jnp.ones((K,), jnp.float32)
    zeros_deg = jnp.zeros((ROWS_W,), jnp.float32)
    zeros_rows = jnp.zeros((ROWS_W, D_HID), jnp.float32)

    deg = _sc_degree(dst, ones_k, zeros_deg)

    xw = pl.pallas_call(_tc_xw_body, out_shape=_f32((N_NODES, D_HID)))(x, W1)

    dinv, xws = pl.pallas_call(
        _tc_prescale_body,
        out_shape=(_f32((N_PAD, 1)), _f32((N_NODES, D_HID))),
    )(deg, xw)

    p1 = _sc_aggregate(xws, src, dst, zeros_rows)

    h, hs = pl.pallas_call(
        _tc_mid_body,
        out_shape=(_f32((N_NODES, D_HID)), _f32((N_NODES, D_HID))),
    )(p1, xw, dinv, b1.reshape(1, D_HID))

    p2 = _sc_aggregate(hs, src, dst, zeros_rows)

    out = pl.pallas_call(
        _tc_final_body,
        out_shape=_f32((N_NODES, 2)),
    )(p2, h, dinv, W2, b2.reshape(1, 2))
    return out


# speedup vs baseline: 38.1770x; 1.0776x over previous
import functools

import jax
import jax.numpy as jnp
from jax import lax
from jax.experimental import pallas as pl
from jax.experimental.pallas import tpu as pltpu
from jax.experimental.pallas import tpu_sc as plsc

N_NODES = 10000
N_PAD = 10240
N_EDGES = 320000
D_IN = 128
D_HID = 16

NC = 2
NS = 16
NW = NC * NS
E_W = N_EDGES // NW
K = 125
WIN = E_W // K
ROWS_W = N_PAD // NS

_MESH = plsc.VectorSubcoreMesh(core_axis_name="c", subcore_axis_name="s")
_SC_PARAMS = pltpu.CompilerParams(use_tc_tiling_on_sc=False)


@functools.partial(
    pl.kernel,
    out_type=jax.ShapeDtypeStruct((NC, N_PAD), jnp.float32),
    mesh=_MESH,
    scratch_types=[
        pltpu.VMEM((WIN, K), jnp.int32),
        pltpu.VMEM((K,), jnp.float32),
        pltpu.VMEM_SHARED((N_PAD,), jnp.float32),
    ],
    compiler_params=_SC_PARAMS,
)
def _sc_degree(dst_hbm, ones_hbm, zeros_hbm, out_hbm, idx_v, ones_v, acc_sh):
    c = lax.axis_index("c")
    s = lax.axis_index("s")
    w = c * NS + s
    pltpu.sync_copy(ones_hbm, ones_v)
    pltpu.sync_copy(zeros_hbm.at[pl.ds(0, ROWS_W)],
                    acc_sh.at[pl.ds(s * ROWS_W, ROWS_W)])
    plsc.subcore_barrier()
    pltpu.sync_copy(dst_hbm.at[pl.ds(w * WIN, WIN)], idx_v)

    @pl.loop(0, WIN)
    def _(j):
        pltpu.sync_copy(ones_v, acc_sh.at[idx_v.at[j]], add=True)

    plsc.subcore_barrier()
    pltpu.sync_copy(acc_sh.at[pl.ds(s * ROWS_W, ROWS_W)],
                    out_hbm.at[c].at[pl.ds(s * ROWS_W, ROWS_W)])


@functools.partial(
    pl.kernel,
    out_type=jax.ShapeDtypeStruct((NC, N_PAD, D_HID), jnp.float32),
    mesh=_MESH,
    scratch_types=[
        pltpu.VMEM((WIN, K), jnp.int32),
        pltpu.VMEM((WIN, K), jnp.int32),
        pltpu.VMEM((K, D_HID), jnp.float32),
        pltpu.VMEM((K, D_HID), jnp.float32),
        pltpu.VMEM_SHARED((N_PAD, D_HID), jnp.float32),
        pltpu.SemaphoreType.DMA,
        pltpu.SemaphoreType.DMA,
    ],
    compiler_params=_SC_PARAMS,
)
def _sc_aggregate(table_hbm, src_hbm, dst_hbm, zeros_hbm, out_hbm,
                  src_v, dst_v, rows_a, rows_b, acc_sh, sem_a, sem_b):
    c = lax.axis_index("c")
    s = lax.axis_index("s")
    w = c * NS + s
    pltpu.sync_copy(zeros_hbm, acc_sh.at[pl.ds(s * ROWS_W, ROWS_W)])
    plsc.subcore_barrier()
    pltpu.sync_copy(src_hbm.at[pl.ds(w * WIN, WIN)], src_v)
    pltpu.sync_copy(dst_hbm.at[pl.ds(w * WIN, WIN)], dst_v)

    pltpu.async_copy(table_hbm.at[src_v.at[0]], rows_a, sem_a)

    @pl.loop(0, WIN, step=2)
    def _(j):
        pltpu.make_async_copy(table_hbm.at[src_v.at[j]], rows_a, sem_a).wait()
        pltpu.async_copy(table_hbm.at[src_v.at[j + 1]], rows_b, sem_b)
        pltpu.sync_copy(rows_a, acc_sh.at[dst_v.at[j]], add=True)

        pltpu.make_async_copy(table_hbm.at[src_v.at[j + 1]], rows_b, sem_b).wait()

        @pl.when(j + 2 < WIN)
        def _():
            pltpu.async_copy(table_hbm.at[src_v.at[j + 2]], rows_a, sem_a)

        pltpu.sync_copy(rows_b, acc_sh.at[dst_v.at[j + 1]], add=True)

    plsc.subcore_barrier()
    pltpu.sync_copy(acc_sh.at[pl.ds(s * ROWS_W, ROWS_W)],
                    out_hbm.at[c].at[pl.ds(s * ROWS_W, ROWS_W)])


def _tc_head_body(x_ref, w1_ref, deg_ref, xw_ref, dinv_ref, xws_ref):
    xw = jnp.dot(x_ref[...], w1_ref[...], preferred_element_type=jnp.float32)
    xw_ref[...] = xw
    deg = deg_ref[0:1, :] + deg_ref[1:2, :] + 1.0
    dinv = 1.0 / jnp.sqrt(deg)
    dcol = dinv.reshape(N_PAD, 1)
    dinv_ref[...] = dcol
    xws_ref[...] = xw * dcol[0:N_NODES, :]


def _tc_mid_body(p_ref, xw_ref, dinv_ref, b1_ref, h_ref, hs_ref):
    dc = dinv_ref[0:N_NODES, :]
    agg = p_ref[0, 0:N_NODES, :] + p_ref[1, 0:N_NODES, :]
    pre = dc * agg + dc * dc * xw_ref[...] + b1_ref[...]
    h = jnp.maximum(pre, 0.0)
    h_ref[...] = h
    hs_ref[...] = h * dc


def _tc_final_body(q_ref, h_ref, dinv_ref, w2_ref, b2_ref, out_ref):
    dc = dinv_ref[0:N_NODES, :]
    agg2 = q_ref[0, 0:N_NODES, :] + q_ref[1, 0:N_NODES, :]
    z = dc * agg2 + dc * dc * h_ref[...]
    o = jnp.dot(z, w2_ref[...], preferred_element_type=jnp.float32) + b2_ref[...]
    m = jnp.max(o, axis=1, keepdims=True)
    lse = m + jnp.log(jnp.sum(jnp.exp(o - m), axis=1, keepdims=True))
    out_ref[...] = o - lse


def _f32(shape):
    return jax.ShapeDtypeStruct(shape, jnp.float32)


def kernel(x, edge_index, W1, b1, W2, b2):
    src = edge_index[0].astype(jnp.int32).reshape(NW * WIN, K)
    dst = edge_index[1].astype(jnp.int32).reshape(NW * WIN, K)
    ones_k = jnp.ones((K,), jnp.float32)
    zeros_deg = jnp.zeros((ROWS_W,), jnp.float32)
    zeros_rows = jnp.zeros((ROWS_W, D_HID), jnp.float32)

    deg = _sc_degree(dst, ones_k, zeros_deg)

    xw, dinv, xws = pl.pallas_call(
        _tc_head_body,
        out_shape=(_f32((N_NODES, D_HID)), _f32((N_PAD, 1)),
                   _f32((N_NODES, D_HID))),
    )(x, W1, deg)

    p1 = _sc_aggregate(xws, src, dst, zeros_rows)

    h, hs = pl.pallas_call(
        _tc_mid_body,
        out_shape=(_f32((N_NODES, D_HID)), _f32((N_NODES, D_HID))),
    )(p1, xw, dinv, b1.reshape(1, D_HID))

    p2 = _sc_aggregate(hs, src, dst, zeros_rows)

    out = pl.pallas_call(
        _tc_final_body,
        out_shape=_f32((N_NODES, 2)),
    )(p2, h, dinv, W2, b2.reshape(1, 2))
    return out


# speedup vs baseline: 57.6262x; 1.5094x over previous
import functools

import jax
import jax.numpy as jnp
from jax import lax
from jax.experimental import pallas as pl
from jax.experimental.pallas import tpu as pltpu
from jax.experimental.pallas import tpu_sc as plsc

N_NODES = 10000
N_PAD = 10240
N_EDGES = 320000
D_IN = 128
D_HID = 16

NC = 2
NS = 16
NW = NC * NS
E_W = N_EDGES // NW
K = 125
WIN = E_W // K
ROWS_W = N_PAD // NS

_MESH = plsc.VectorSubcoreMesh(core_axis_name="c", subcore_axis_name="s")
_SC_PARAMS = pltpu.CompilerParams(use_tc_tiling_on_sc=False)


@functools.partial(
    pl.kernel,
    out_type=jax.ShapeDtypeStruct((NC, N_PAD), jnp.float32),
    mesh=_MESH,
    scratch_types=[
        pltpu.VMEM((WIN, K), jnp.int32),
        pltpu.VMEM((K,), jnp.float32),
        pltpu.VMEM_SHARED((N_PAD,), jnp.float32),
    ],
    compiler_params=_SC_PARAMS,
)
def _sc_degree(dst_hbm, ones_hbm, zeros_hbm, out_hbm, idx_v, ones_v, acc_sh):
    c = lax.axis_index("c")
    s = lax.axis_index("s")
    w = c * NS + s
    pltpu.sync_copy(ones_hbm, ones_v)
    pltpu.sync_copy(zeros_hbm.at[pl.ds(0, ROWS_W)],
                    acc_sh.at[pl.ds(s * ROWS_W, ROWS_W)])
    plsc.subcore_barrier()
    pltpu.sync_copy(dst_hbm.at[pl.ds(w * WIN, WIN)], idx_v)

    @pl.loop(0, WIN)
    def _(j):
        pltpu.sync_copy(ones_v, acc_sh.at[idx_v.at[j]], add=True)

    plsc.subcore_barrier()
    pltpu.sync_copy(acc_sh.at[pl.ds(s * ROWS_W, ROWS_W)],
                    out_hbm.at[c].at[pl.ds(s * ROWS_W, ROWS_W)])


@functools.partial(
    pl.kernel,
    out_type=jax.ShapeDtypeStruct((NC, N_PAD, D_HID), jnp.float32),
    mesh=_MESH,
    scratch_types=[
        pltpu.VMEM((WIN, K), jnp.int32),
        pltpu.VMEM((WIN, K), jnp.int32),
        pltpu.VMEM((K, D_HID), jnp.float32),
        pltpu.VMEM((K, D_HID), jnp.float32),
        pltpu.VMEM_SHARED((N_PAD, D_HID), jnp.float32),
        pltpu.VMEM_SHARED((N_PAD, D_HID), jnp.float32),
        pltpu.SemaphoreType.DMA,
        pltpu.SemaphoreType.DMA,
    ],
    compiler_params=_SC_PARAMS,
)
def _sc_aggregate(table_hbm, src_hbm, dst_hbm, zeros_hbm, out_hbm,
                  src_v, dst_v, rows_a, rows_b, tab_sh, acc_sh, sem_a, sem_b):
    c = lax.axis_index("c")
    s = lax.axis_index("s")
    w = c * NS + s
    pltpu.sync_copy(table_hbm.at[pl.ds(s * ROWS_W, ROWS_W)],
                    tab_sh.at[pl.ds(s * ROWS_W, ROWS_W)])
    pltpu.sync_copy(zeros_hbm, acc_sh.at[pl.ds(s * ROWS_W, ROWS_W)])
    pltpu.sync_copy(src_hbm.at[pl.ds(w * WIN, WIN)], src_v)
    pltpu.sync_copy(dst_hbm.at[pl.ds(w * WIN, WIN)], dst_v)
    plsc.subcore_barrier()

    pltpu.async_copy(tab_sh.at[src_v.at[0]], rows_a, sem_a)

    @pl.loop(0, WIN, step=2)
    def _(j):
        pltpu.make_async_copy(tab_sh.at[src_v.at[j]], rows_a, sem_a).wait()
        pltpu.async_copy(tab_sh.at[src_v.at[j + 1]], rows_b, sem_b)
        pltpu.sync_copy(rows_a, acc_sh.at[dst_v.at[j]], add=True)

        pltpu.make_async_copy(tab_sh.at[src_v.at[j + 1]], rows_b, sem_b).wait()

        @pl.when(j + 2 < WIN)
        def _():
            pltpu.async_copy(tab_sh.at[src_v.at[j + 2]], rows_a, sem_a)

        pltpu.sync_copy(rows_b, acc_sh.at[dst_v.at[j + 1]], add=True)

    plsc.subcore_barrier()
    pltpu.sync_copy(acc_sh.at[pl.ds(s * ROWS_W, ROWS_W)],
                    out_hbm.at[c].at[pl.ds(s * ROWS_W, ROWS_W)])


def _tc_head_body(x_ref, w1_ref, deg_ref, xw_ref, dinv_ref, xws_ref):
    xw = jnp.dot(x_ref[...], w1_ref[...], preferred_element_type=jnp.float32)
    xw_ref[...] = xw
    deg = deg_ref[0:1, :] + deg_ref[1:2, :] + 1.0
    dinv = 1.0 / jnp.sqrt(deg)
    dcol = dinv.reshape(N_PAD, 1)
    dinv_ref[...] = dcol
    xws_ref[0:N_NODES, :] = xw * dcol[0:N_NODES, :]
    xws_ref[N_NODES:N_PAD, :] = jnp.zeros((N_PAD - N_NODES, D_HID), jnp.float32)


def _tc_mid_body(p_ref, xw_ref, dinv_ref, b1_ref, h_ref, hs_ref):
    dc = dinv_ref[0:N_NODES, :]
    agg = p_ref[0, 0:N_NODES, :] + p_ref[1, 0:N_NODES, :]
    pre = dc * agg + dc * dc * xw_ref[...] + b1_ref[...]
    h = jnp.maximum(pre, 0.0)
    h_ref[...] = h
    hs_ref[0:N_NODES, :] = h * dc
    hs_ref[N_NODES:N_PAD, :] = jnp.zeros((N_PAD - N_NODES, D_HID), jnp.float32)


def _tc_final_body(q_ref, h_ref, dinv_ref, w2_ref, b2_ref, out_ref):
    dc = dinv_ref[0:N_NODES, :]
    agg2 = q_ref[0, 0:N_NODES, :] + q_ref[1, 0:N_NODES, :]
    z = dc * agg2 + dc * dc * h_ref[...]
    o = jnp.dot(z, w2_ref[...], preferred_element_type=jnp.float32) + b2_ref[...]
    m = jnp.max(o, axis=1, keepdims=True)
    lse = m + jnp.log(jnp.sum(jnp.exp(o - m), axis=1, keepdims=True))
    out_ref[...] = o - lse


def _f32(shape):
    return jax.ShapeDtypeStruct(shape, jnp.float32)


def kernel(x, edge_index, W1, b1, W2, b2):
    src = edge_index[0].astype(jnp.int32).reshape(NW * WIN, K)
    dst = edge_index[1].astype(jnp.int32).reshape(NW * WIN, K)
    ones_k = jnp.ones((K,), jnp.float32)
    zeros_deg = jnp.zeros((ROWS_W,), jnp.float32)
    zeros_rows = jnp.zeros((ROWS_W, D_HID), jnp.float32)

    deg = _sc_degree(dst, ones_k, zeros_deg)

    xw, dinv, xws = pl.pallas_call(
        _tc_head_body,
        out_shape=(_f32((N_NODES, D_HID)), _f32((N_PAD, 1)),
                   _f32((N_PAD, D_HID))),
    )(x, W1, deg)

    p1 = _sc_aggregate(xws, src, dst, zeros_rows)

    h, hs = pl.pallas_call(
        _tc_mid_body,
        out_shape=(_f32((N_NODES, D_HID)), _f32((N_PAD, D_HID))),
    )(p1, xw, dinv, b1.reshape(1, D_HID))

    p2 = _sc_aggregate(hs, src, dst, zeros_rows)

    out = pl.pallas_call(
        _tc_final_body,
        out_shape=_f32((N_NODES, 2)),
    )(p2, h, dinv, W2, b2.reshape(1, 2))
    return out


# speedup vs baseline: 58.3003x; 1.0117x over previous
import functools

import jax
import jax.numpy as jnp
from jax import lax
from jax.experimental import pallas as pl
from jax.experimental.pallas import tpu as pltpu
from jax.experimental.pallas import tpu_sc as plsc

N_NODES = 10000
N_PAD = 10240
N_EDGES = 320000
D_IN = 128
D_HID = 16

NC = 2
NS = 16
NW = NC * NS
K = 128
WIN = 80
E_PAD = NW * WIN * K
ROWS_W = N_PAD // NS

_MESH = plsc.VectorSubcoreMesh(core_axis_name="c", subcore_axis_name="s")
_SC_PARAMS = pltpu.CompilerParams(use_tc_tiling_on_sc=False)


@functools.partial(
    pl.kernel,
    out_type=jax.ShapeDtypeStruct((NC, N_PAD), jnp.float32),
    mesh=_MESH,
    scratch_types=[
        pltpu.VMEM((WIN, K), jnp.int32),
        pltpu.VMEM((K,), jnp.float32),
        pltpu.VMEM_SHARED((N_PAD,), jnp.float32),
    ],
    compiler_params=_SC_PARAMS,
)
def _sc_degree(dst_hbm, ones_hbm, zeros_hbm, out_hbm, idx_v, ones_v, acc_sh):
    c = lax.axis_index("c")
    s = lax.axis_index("s")
    w = c * NS + s
    pltpu.sync_copy(ones_hbm, ones_v)
    pltpu.sync_copy(zeros_hbm.at[pl.ds(0, ROWS_W)],
                    acc_sh.at[pl.ds(s * ROWS_W, ROWS_W)])
    plsc.subcore_barrier()
    pltpu.sync_copy(dst_hbm.at[pl.ds(w * WIN, WIN)], idx_v)

    @pl.loop(0, WIN)
    def _(j):
        pltpu.sync_copy(ones_v, acc_sh.at[idx_v.at[j]], add=True)

    plsc.subcore_barrier()
    pltpu.sync_copy(acc_sh.at[pl.ds(s * ROWS_W, ROWS_W)],
                    out_hbm.at[c].at[pl.ds(s * ROWS_W, ROWS_W)])


@functools.partial(
    pl.kernel,
    out_type=jax.ShapeDtypeStruct((NC, N_PAD, D_HID), jnp.float32),
    mesh=_MESH,
    scratch_types=[
        pltpu.VMEM((WIN, K), jnp.int32),
        pltpu.VMEM((WIN, K), jnp.int32),
        pltpu.VMEM((K, D_HID), jnp.float32),
        pltpu.VMEM((K, D_HID), jnp.float32),
        pltpu.VMEM_SHARED((N_PAD, D_HID), jnp.float32),
        pltpu.VMEM_SHARED((N_PAD, D_HID), jnp.float32),
        pltpu.SemaphoreType.DMA,
        pltpu.SemaphoreType.DMA,
    ],
    compiler_params=_SC_PARAMS,
)
def _sc_aggregate(table_hbm, src_hbm, dst_hbm, zeros_hbm, out_hbm,
                  src_v, dst_v, rows_a, rows_b, tab_sh, acc_sh, sem_a, sem_b):
    c = lax.axis_index("c")
    s = lax.axis_index("s")
    w = c * NS + s
    pltpu.sync_copy(table_hbm.at[pl.ds(s * ROWS_W, ROWS_W)],
                    tab_sh.at[pl.ds(s * ROWS_W, ROWS_W)])
    pltpu.sync_copy(zeros_hbm, acc_sh.at[pl.ds(s * ROWS_W, ROWS_W)])
    pltpu.sync_copy(src_hbm.at[pl.ds(w * WIN, WIN)], src_v)
    pltpu.sync_copy(dst_hbm.at[pl.ds(w * WIN, WIN)], dst_v)
    plsc.subcore_barrier()

    pltpu.async_copy(tab_sh.at[src_v.at[0]], rows_a, sem_a)

    @pl.loop(0, WIN, step=2)
    def _(j):
        pltpu.make_async_copy(tab_sh.at[src_v.at[j]], rows_a, sem_a).wait()
        pltpu.async_copy(tab_sh.at[src_v.at[j + 1]], rows_b, sem_b)
        pltpu.sync_copy(rows_a, acc_sh.at[dst_v.at[j]], add=True)

        pltpu.make_async_copy(tab_sh.at[src_v.at[j + 1]], rows_b, sem_b).wait()

        @pl.when(j + 2 < WIN)
        def _():
            pltpu.async_copy(tab_sh.at[src_v.at[j + 2]], rows_a, sem_a)

        pltpu.sync_copy(rows_b, acc_sh.at[dst_v.at[j + 1]], add=True)

    plsc.subcore_barrier()
    pltpu.sync_copy(acc_sh.at[pl.ds(s * ROWS_W, ROWS_W)],
                    out_hbm.at[c].at[pl.ds(s * ROWS_W, ROWS_W)])


def _tc_head_body(x_ref, w1_ref, deg_ref, xw_ref, dinv_ref, xws_ref):
    xw = jnp.dot(x_ref[...], w1_ref[...], preferred_element_type=jnp.float32)
    xw_ref[...] = xw
    deg = deg_ref[0:1, :] + deg_ref[1:2, :] + 1.0
    dinv = 1.0 / jnp.sqrt(deg)
    dcol = dinv.reshape(N_PAD, 1)
    dinv_ref[...] = dcol
    xws_ref[0:N_NODES, :] = xw * dcol[0:N_NODES, :]
    xws_ref[N_NODES:N_PAD, :] = jnp.zeros((N_PAD - N_NODES, D_HID), jnp.float32)


def _tc_mid_body(p_ref, xw_ref, dinv_ref, b1_ref, h_ref, hs_ref):
    dc = dinv_ref[0:N_NODES, :]
    agg = p_ref[0, 0:N_NODES, :] + p_ref[1, 0:N_NODES, :]
    pre = dc * agg + dc * dc * xw_ref[...] + b1_ref[...]
    h = jnp.maximum(pre, 0.0)
    h_ref[...] = h
    hs_ref[0:N_NODES, :] = h * dc
    hs_ref[N_NODES:N_PAD, :] = jnp.zeros((N_PAD - N_NODES, D_HID), jnp.float32)


def _tc_final_body(q_ref, h_ref, dinv_ref, w2_ref, b2_ref, out_ref):
    dc = dinv_ref[0:N_NODES, :]
    agg2 = q_ref[0, 0:N_NODES, :] + q_ref[1, 0:N_NODES, :]
    z = dc * agg2 + dc * dc * h_ref[...]
    o = jnp.dot(z, w2_ref[...], preferred_element_type=jnp.float32) + b2_ref[...]
    m = jnp.max(o, axis=1, keepdims=True)
    lse = m + jnp.log(jnp.sum(jnp.exp(o - m), axis=1, keepdims=True))
    out_ref[...] = o - lse


def _f32(shape):
    return jax.ShapeDtypeStruct(shape, jnp.float32)


def kernel(x, edge_index, W1, b1, W2, b2):
    pads = (jnp.arange(E_PAD - N_EDGES, dtype=jnp.int32) % (N_PAD - N_NODES)
            ) + N_NODES
    src = jnp.concatenate([edge_index[0].astype(jnp.int32), pads]
                          ).reshape(NW * WIN, K)
    dst = jnp.concatenate([edge_index[1].astype(jnp.int32), pads]
                          ).reshape(NW * WIN, K)
    ones_k = jnp.ones((K,), jnp.float32)
    zeros_deg = jnp.zeros((ROWS_W,), jnp.float32)
    zeros_rows = jnp.zeros((ROWS_W, D_HID), jnp.float32)

    deg = _sc_degree(dst, ones_k, zeros_deg)

    xw, dinv, xws = pl.pallas_call(
        _tc_head_body,
        out_shape=(_f32((N_NODES, D_HID)), _f32((N_PAD, 1)),
                   _f32((N_PAD, D_HID))),
    )(x, W1, deg)

    p1 = _sc_aggregate(xws, src, dst, zeros_rows)

    h, hs = pl.pallas_call(
        _tc_mid_body,
        out_shape=(_f32((N_NODES, D_HID)), _f32((N_PAD, D_HID))),
    )(p1, xw, dinv, b1.reshape(1, D_HID))

    p2 = _sc_aggregate(hs, src, dst, zeros_rows)

    out = pl.pallas_call(
        _tc_final_body,
        out_shape=_f32((N_NODES, 2)),
    )(p2, h, dinv, W2, b2.reshape(1, 2))
    return out


# speedup vs baseline: 86.6332x; 1.4860x over previous
import functools

import jax
import jax.numpy as jnp
from jax import lax
from jax.experimental import pallas as pl
from jax.experimental.pallas import tpu as pltpu
from jax.experimental.pallas import tpu_sc as plsc

N_NODES = 10000
N_PAD = 10240
N_EDGES = 320000
D_IN = 128
D_HID = 16

NC = 2
NS = 16
NW = NC * NS
K = 128
WIN = 80
E_PAD = NW * WIN * K
ROWS_W = N_PAD // NS
WROWS = N_PAD // 8
WREAL = N_NODES // 8

_MESH = plsc.VectorSubcoreMesh(core_axis_name="c", subcore_axis_name="s")
_SC_PARAMS = pltpu.CompilerParams(use_tc_tiling_on_sc=False)


@functools.partial(
    pl.kernel,
    out_type=jax.ShapeDtypeStruct((NC, N_PAD), jnp.float32),
    mesh=_MESH,
    scratch_types=[
        pltpu.VMEM((WIN, K), jnp.int32),
        pltpu.VMEM((K,), jnp.float32),
        pltpu.VMEM_SHARED((N_PAD,), jnp.float32),
    ],
    compiler_params=_SC_PARAMS,
)
def _sc_degree(edges_hbm, ones_hbm, zeros_hbm, out_hbm, idx_v, ones_v, acc_sh):
    c = lax.axis_index("c")
    s = lax.axis_index("s")
    w = c * NS + s
    pltpu.sync_copy(ones_hbm, ones_v)
    pltpu.sync_copy(zeros_hbm.at[pl.ds(0, ROWS_W)],
                    acc_sh.at[pl.ds(s * ROWS_W, ROWS_W)])
    pltpu.sync_copy(edges_hbm.at[1].at[pl.ds(w * WIN, WIN)], idx_v)
    plsc.subcore_barrier()

    @pl.loop(0, WIN)
    def _(j):
        pltpu.sync_copy(ones_v, acc_sh.at[idx_v.at[j]], add=True)

    plsc.subcore_barrier()
    pltpu.sync_copy(acc_sh.at[pl.ds(s * ROWS_W, ROWS_W)],
                    out_hbm.at[c].at[pl.ds(s * ROWS_W, ROWS_W)])


@functools.partial(
    pl.kernel,
    out_type=jax.ShapeDtypeStruct((NC, N_PAD, D_HID), jnp.float32),
    mesh=_MESH,
    scratch_types=[
        pltpu.VMEM((WIN, K), jnp.int32),
        pltpu.VMEM((WIN, K), jnp.int32),
        pltpu.VMEM((K, D_HID), jnp.float32),
        pltpu.VMEM((K, D_HID), jnp.float32),
        pltpu.VMEM_SHARED((N_PAD, D_HID), jnp.float32),
        pltpu.VMEM_SHARED((N_PAD, D_HID), jnp.float32),
        pltpu.SemaphoreType.DMA,
        pltpu.SemaphoreType.DMA,
    ],
    compiler_params=_SC_PARAMS,
)
def _sc_aggregate(table_hbm, edges_hbm, zeros_hbm, out_hbm,
                  src_v, dst_v, rows_a, rows_b, tab_sh, acc_sh, sem_a, sem_b):
    c = lax.axis_index("c")
    s = lax.axis_index("s")
    w = c * NS + s
    pltpu.sync_copy(table_hbm.at[pl.ds(s * ROWS_W, ROWS_W)],
                    tab_sh.at[pl.ds(s * ROWS_W, ROWS_W)])
    pltpu.sync_copy(zeros_hbm, acc_sh.at[pl.ds(s * ROWS_W, ROWS_W)])
    pltpu.sync_copy(edges_hbm.at[0].at[pl.ds(w * WIN, WIN)], src_v)
    pltpu.sync_copy(edges_hbm.at[1].at[pl.ds(w * WIN, WIN)], dst_v)
    plsc.subcore_barrier()

    pltpu.async_copy(tab_sh.at[src_v.at[0]], rows_a, sem_a)

    @pl.loop(0, WIN, step=2)
    def _(j):
        pltpu.make_async_copy(tab_sh.at[src_v.at[j]], rows_a, sem_a).wait()
        pltpu.async_copy(tab_sh.at[src_v.at[j + 1]], rows_b, sem_b)
        pltpu.sync_copy(rows_a, acc_sh.at[dst_v.at[j]], add=True)

        pltpu.make_async_copy(tab_sh.at[src_v.at[j + 1]], rows_b, sem_b).wait()

        @pl.when(j + 2 < WIN)
        def _():
            pltpu.async_copy(tab_sh.at[src_v.at[j + 2]], rows_a, sem_a)

        pltpu.sync_copy(rows_b, acc_sh.at[dst_v.at[j + 1]], add=True)

    plsc.subcore_barrier()
    pltpu.sync_copy(acc_sh.at[pl.ds(s * ROWS_W, ROWS_W)],
                    out_hbm.at[c].at[pl.ds(s * ROWS_W, ROWS_W)])


def _tc_head_body(x2_ref, w1k_ref, deg_ref, e8_ref, xwsw_ref, dinvw_ref):
    d8 = deg_ref[0] + deg_ref[1] + 1.0
    dinv8 = 1.0 / jnp.sqrt(d8)
    dinvw = jnp.dot(dinv8, e8_ref[...],
                    preferred_element_type=jnp.float32)
    dinvw_ref[...] = dinvw
    xww = jnp.dot(x2_ref[...], w1k_ref[...],
                  preferred_element_type=jnp.float32)
    xwsw_ref[0:WREAL, :] = xww * dinvw[0:WREAL, :]
    xwsw_ref[WREAL:WROWS, :] = jnp.zeros((WROWS - WREAL, 8 * D_HID),
                                         jnp.float32)


def _tc_mid_body(p_ref, xwsw_ref, dinvw_ref, b1w_ref, hsw_ref):
    agg = p_ref[0] + p_ref[1]
    t = dinvw_ref[...] * (agg + xwsw_ref[...]) + b1w_ref[...]
    hsw = dinvw_ref[...] * jnp.maximum(t, 0.0)
    hsw_ref[0:WREAL, :] = hsw[0:WREAL, :]
    hsw_ref[WREAL:WROWS, :] = jnp.zeros((WROWS - WREAL, 8 * D_HID),
                                        jnp.float32)


def _tc_final_body(q_ref, hsw_ref, dinvw_ref, w2k_ref, b2w_ref, out_ref):
    z = dinvw_ref[...] * (q_ref[0] + q_ref[1] + hsw_ref[...])
    o = jnp.dot(z, w2k_ref[...],
                preferred_element_type=jnp.float32) + b2w_ref[...]
    lane = lax.broadcasted_iota(jnp.int32, (1, 16), 1)
    even = (lane % 2) == 0
    partner = jnp.where(even, jnp.roll(o, -1, axis=1), jnp.roll(o, 1, axis=1))
    m = jnp.maximum(o, partner)
    e = jnp.exp(o - m)
    pe = jnp.where(even, jnp.roll(e, -1, axis=1), jnp.roll(e, 1, axis=1))
    out_ref[...] = o - (m + jnp.log(e + pe))


def _f32(shape):
    return jax.ShapeDtypeStruct(shape, jnp.float32)


def kernel(x, edge_index, W1, b1, W2, b2):
    f32 = jnp.float32
    ei = edge_index.astype(jnp.int32).reshape(2, N_EDGES // K, K)
    padrow = ((lax.iota(jnp.int32, (E_PAD - N_EDGES)) % (N_PAD - N_NODES))
              + N_NODES).reshape(1, (E_PAD - N_EDGES) // K, K)
    edges = jnp.concatenate(
        [ei, jnp.concatenate([padrow, padrow], axis=0)], axis=1)

    ones_k = jnp.ones((K,), f32)
    zeros_deg = jnp.zeros((ROWS_W,), f32)
    zeros_rows = jnp.zeros((ROWS_W, D_HID), f32)
    e8 = jnp.kron(jnp.eye(8, dtype=f32), jnp.ones((1, D_HID), f32))
    w1k = jnp.kron(jnp.eye(8, dtype=f32), W1)
    w2k = jnp.kron(jnp.eye(8, dtype=f32), W2)
    b1w = jnp.tile(b1, 8).reshape(1, 8 * D_HID)
    b2w = jnp.tile(b2, 8).reshape(1, 16)

    deg = _sc_degree(edges, ones_k, zeros_deg)

    xwsw, dinvw = pl.pallas_call(
        _tc_head_body,
        out_shape=(_f32((WROWS, 8 * D_HID)), _f32((WROWS, 8 * D_HID))),
    )(x.reshape(WREAL, 8 * D_IN), w1k, deg.reshape(2, WROWS, 8), e8)

    p = _sc_aggregate(xwsw.reshape(N_PAD, D_HID), edges, zeros_rows)

    hsw = pl.pallas_call(
        _tc_mid_body,
        out_shape=_f32((WROWS, 8 * D_HID)),
    )(p.reshape(NC, WROWS, 8 * D_HID), xwsw, dinvw, b1w)

    q = _sc_aggregate(hsw.reshape(N_PAD, D_HID), edges, zeros_rows)

    outw = pl.pallas_call(
        _tc_final_body,
        out_shape=_f32((WROWS, 16)),
    )(q.reshape(NC, WROWS, 8 * D_HID), hsw, dinvw, w2k, b2w)

    return outw.reshape(N_PAD, 2)[0:N_NODES]


# speedup vs baseline: 93.9146x; 1.0840x over previous
import functools

import jax
import jax.numpy as jnp
from jax import lax
from jax.experimental import pallas as pl
from jax.experimental.pallas import tpu as pltpu
from jax.experimental.pallas import tpu_sc as plsc

N_NODES = 10000
N_PAD = 10240
N_EDGES = 320000
D_IN = 128
D_HID = 16

NC = 2
NS = 16
NW = NC * NS
K = 128
WIN = 80
E_PAD = NW * WIN * K
ROWS_W = N_PAD // NS
WROWS = N_PAD // 8
WREAL = N_NODES // 8

_MESH = plsc.VectorSubcoreMesh(core_axis_name="c", subcore_axis_name="s")
_SC_PARAMS = pltpu.CompilerParams(use_tc_tiling_on_sc=False)


@functools.partial(
    pl.kernel,
    out_type=jax.ShapeDtypeStruct((NC, N_PAD), jnp.float32),
    mesh=_MESH,
    scratch_types=[
        pltpu.VMEM((WIN, K), jnp.int32),
        pltpu.VMEM((K,), jnp.float32),
        pltpu.VMEM_SHARED((N_PAD,), jnp.float32),
    ],
    compiler_params=_SC_PARAMS,
)
def _sc_degree(edges_hbm, ones_hbm, zeros_hbm, out_hbm, idx_v, ones_v, acc_sh):
    c = lax.axis_index("c")
    s = lax.axis_index("s")
    w = c * NS + s
    pltpu.sync_copy(ones_hbm, ones_v)
    pltpu.sync_copy(zeros_hbm.at[pl.ds(0, ROWS_W)],
                    acc_sh.at[pl.ds(s * ROWS_W, ROWS_W)])
    pltpu.sync_copy(edges_hbm.at[1].at[pl.ds(w * WIN, WIN)], idx_v)
    plsc.subcore_barrier()

    @pl.loop(0, WIN)
    def _(j):
        pltpu.sync_copy(ones_v, acc_sh.at[idx_v.at[j]], add=True)

    plsc.subcore_barrier()
    pltpu.sync_copy(acc_sh.at[pl.ds(s * ROWS_W, ROWS_W)],
                    out_hbm.at[c].at[pl.ds(s * ROWS_W, ROWS_W)])


@functools.partial(
    pl.kernel,
    out_type=jax.ShapeDtypeStruct((NC, N_PAD, D_HID), jnp.float32),
    mesh=_MESH,
    scratch_types=[
        pltpu.VMEM((WIN, K), jnp.int32),
        pltpu.VMEM((WIN, K), jnp.int32),
        pltpu.VMEM((K, D_HID), jnp.float32),
        pltpu.VMEM((K, D_HID), jnp.float32),
        pltpu.VMEM((K, D_HID), jnp.float32),
        pltpu.VMEM((K, D_HID), jnp.float32),
        pltpu.VMEM_SHARED((N_PAD, D_HID), jnp.float32),
        pltpu.VMEM_SHARED((N_PAD, D_HID), jnp.float32),
        pltpu.SemaphoreType.DMA,
        pltpu.SemaphoreType.DMA,
        pltpu.SemaphoreType.DMA,
    ],
    compiler_params=_SC_PARAMS,
)
def _sc_aggregate(table_hbm, edges_hbm, zeros_hbm, out_hbm,
                  src_v, dst_v, rows_a, rows_b, rows_c, rows_d,
                  tab_sh, acc_sh, sem_p, sem_g, sem_s):
    c = lax.axis_index("c")
    s = lax.axis_index("s")
    w = c * NS + s
    prologue = [
        (table_hbm.at[pl.ds(s * ROWS_W, ROWS_W)],
         tab_sh.at[pl.ds(s * ROWS_W, ROWS_W)]),
        (zeros_hbm, acc_sh.at[pl.ds(s * ROWS_W, ROWS_W)]),
        (edges_hbm.at[0].at[pl.ds(w * WIN, WIN)], src_v),
        (edges_hbm.at[1].at[pl.ds(w * WIN, WIN)], dst_v),
    ]
    for a, b in prologue:
        pltpu.async_copy(a, b, sem_p)
    for a, b in prologue:
        pltpu.make_async_copy(a, b, sem_p).wait()
    plsc.subcore_barrier()

    bufs = (rows_a, rows_b, rows_c, rows_d)
    for b, buf in enumerate(bufs):
        pltpu.async_copy(tab_sh.at[src_v.at[b]], buf, sem_g)

    @pl.loop(0, WIN, step=4)
    def _(j):
        for b, buf in enumerate(bufs):
            pltpu.make_async_copy(tab_sh.at[src_v.at[j + b]], buf, sem_g).wait()
            pltpu.async_copy(buf, acc_sh.at[dst_v.at[j + b]], sem_s, add=True)
        for b, buf in enumerate(bufs):
            pltpu.make_async_copy(buf, acc_sh.at[dst_v.at[j + b]], sem_s).wait()

            @pl.when(j + 4 + b < WIN)
            def _():
                pltpu.async_copy(tab_sh.at[src_v.at[j + 4 + b]], buf, sem_g)

    plsc.subcore_barrier()
    pltpu.sync_copy(acc_sh.at[pl.ds(s * ROWS_W, ROWS_W)],
                    out_hbm.at[c].at[pl.ds(s * ROWS_W, ROWS_W)])


def _tc_head_body(x2_ref, w1k_ref, deg_ref, e8_ref, xwsw_ref, dinvw_ref):
    d8 = deg_ref[0] + deg_ref[1] + 1.0
    dinv8 = 1.0 / jnp.sqrt(d8)
    dinvw = jnp.dot(dinv8, e8_ref[...],
                    preferred_element_type=jnp.float32)
    dinvw_ref[...] = dinvw
    xww = jnp.dot(x2_ref[...], w1k_ref[...],
                  preferred_element_type=jnp.float32)
    xwsw_ref[0:WREAL, :] = xww * dinvw[0:WREAL, :]
    xwsw_ref[WREAL:WROWS, :] = jnp.zeros((WROWS - WREAL, 8 * D_HID),
                                         jnp.float32)


def _tc_mid_body(p_ref, xwsw_ref, dinvw_ref, b1w_ref, hsw_ref):
    agg = p_ref[0] + p_ref[1]
    t = dinvw_ref[...] * (agg + xwsw_ref[...]) + b1w_ref[...]
    hsw = dinvw_ref[...] * jnp.maximum(t, 0.0)
    hsw_ref[0:WREAL, :] = hsw[0:WREAL, :]
    hsw_ref[WREAL:WROWS, :] = jnp.zeros((WROWS - WREAL, 8 * D_HID),
                                        jnp.float32)


def _tc_final_body(q_ref, hsw_ref, dinvw_ref, w2k_ref, b2w_ref, out_ref):
    z = dinvw_ref[...] * (q_ref[0] + q_ref[1] + hsw_ref[...])
    o = jnp.dot(z, w2k_ref[...],
                preferred_element_type=jnp.float32) + b2w_ref[...]
    lane = lax.broadcasted_iota(jnp.int32, (1, 16), 1)
    even = (lane % 2) == 0
    partner = jnp.where(even, jnp.roll(o, -1, axis=1), jnp.roll(o, 1, axis=1))
    m = jnp.maximum(o, partner)
    e = jnp.exp(o - m)
    pe = jnp.where(even, jnp.roll(e, -1, axis=1), jnp.roll(e, 1, axis=1))
    out_ref[...] = o - (m + jnp.log(e + pe))


def _f32(shape):
    return jax.ShapeDtypeStruct(shape, jnp.float32)


def kernel(x, edge_index, W1, b1, W2, b2):
    f32 = jnp.float32
    ei = edge_index.astype(jnp.int32).reshape(2, N_EDGES // K, K)
    padrow = ((lax.iota(jnp.int32, (E_PAD - N_EDGES)) % (N_PAD - N_NODES))
              + N_NODES).reshape(1, (E_PAD - N_EDGES) // K, K)
    edges = jnp.concatenate(
        [ei, jnp.concatenate([padrow, padrow], axis=0)], axis=1)

    ones_k = jnp.ones((K,), f32)
    zeros_deg = jnp.zeros((ROWS_W,), f32)
    zeros_rows = jnp.zeros((ROWS_W, D_HID), f32)
    e8 = jnp.kron(jnp.eye(8, dtype=f32), jnp.ones((1, D_HID), f32))
    w1k = jnp.kron(jnp.eye(8, dtype=f32), W1)
    w2k = jnp.kron(jnp.eye(8, dtype=f32), W2)
    b1w = jnp.tile(b1, 8).reshape(1, 8 * D_HID)
    b2w = jnp.tile(b2, 8).reshape(1, 16)

    deg = _sc_degree(edges, ones_k, zeros_deg)

    xwsw, dinvw = pl.pallas_call(
        _tc_head_body,
        out_shape=(_f32((WROWS, 8 * D_HID)), _f32((WROWS, 8 * D_HID))),
    )(x.reshape(WREAL, 8 * D_IN), w1k, deg.reshape(2, WROWS, 8), e8)

    p = _sc_aggregate(xwsw.reshape(N_PAD, D_HID), edges, zeros_rows)

    hsw = pl.pallas_call(
        _tc_mid_body,
        out_shape=_f32((WROWS, 8 * D_HID)),
    )(p.reshape(NC, WROWS, 8 * D_HID), xwsw, dinvw, b1w)

    q = _sc_aggregate(hsw.reshape(N_PAD, D_HID), edges, zeros_rows)

    outw = pl.pallas_call(
        _tc_final_body,
        out_shape=_f32((WROWS, 16)),
    )(q.reshape(NC, WROWS, 8 * D_HID), hsw, dinvw, w2k, b2w)

    return outw[0:WREAL].reshape(N_NODES, 2)
